# baseline (speedup 1.0000x reference)
import functools

import jax
import jax.numpy as jnp
from jax import lax
from jax.experimental import pallas as pl
from jax.experimental.pallas import tpu as pltpu
from jax.experimental.pallas import tpu_sc as plsc

N_NODES = 10000
NP = 10240
E_EDGES = 320000
D_IN = 128
H = 128
O_DIM = 64
O_PAD = 128
U = 50
UP = 64
NC, NS, LANES = 2, 16, 16
NW = NC * NS
CHUNK = 128
NCH = 80
PER_W = NCH * CHUNK
E_PAD = PER_W * NW
TOT_CH = E_PAD // CHUNK
N0 = 96
N1 = 64
assert NS * (N0 + N1) == TOT_CH and N0 % 16 == 0 and N1 % 16 == 0
ROWS_PER_TILE = NP // NS
ZBLK = 128

_PREC = jax.lax.Precision.HIGHEST
_F32 = jnp.float32


def _sc_mesh():
    return plsc.VectorSubcoreMesh(core_axis_name="c", subcore_axis_name="s",
                                  num_cores=NC, num_subcores=NS)


def _deg_call(dst_p):
    @functools.partial(
        pl.kernel,
        out_type=jax.ShapeDtypeStruct((NC, NP, 128), _F32),
        mesh=_sc_mesh(),
        scratch_types=[
            pltpu.VMEM((NCH, CHUNK), jnp.int32),
            pltpu.VMEM((CHUNK, 128), _F32),
            pltpu.VMEM((ZBLK, 128), _F32),
            pltpu.VMEM_SHARED((NP, 128), _F32),
            pltpu.SemaphoreType.DMA,
        ],
    )
    def deg_kernel(dst_hbm, out_hbm, dst_all, ones_v, zero_v, acc_sh, sem):
        cid = lax.axis_index("c")
        sid = lax.axis_index("s")
        tile = cid * NS + sid
        pltpu.sync_copy(dst_hbm.at[tile], dst_all)

        @pl.loop(0, CHUNK)
        def _(i):
            @pl.loop(0, 128, step=LANES)
            def _(j):
                ones_v[i, pl.ds(j, LANES)] = jnp.ones((LANES,), _F32)

        @pl.loop(0, ZBLK)
        def _(i):
            @pl.loop(0, 128, step=LANES)
            def _(j):
                zero_v[i, pl.ds(j, LANES)] = jnp.zeros((LANES,), _F32)

        row0 = sid * ROWS_PER_TILE

        @pl.loop(0, ROWS_PER_TILE, step=ZBLK)
        def _(r):
            pltpu.sync_copy(zero_v, acc_sh.at[pl.ds(row0 + r, ZBLK)])

        plsc.subcore_barrier()

        @pl.loop(0, NCH, step=8)
        def _(c0):
            for k in range(8):
                pltpu.async_copy(ones_v, acc_sh.at[dst_all.at[c0 + k]], sem,
                                 add=True)
            for k in range(8):
                pltpu.make_async_copy(ones_v,
                                      acc_sh.at[dst_all.at[c0 + k]],
                                      sem).wait()

        plsc.subcore_barrier()

        @pl.loop(0, ROWS_PER_TILE, step=ZBLK)
        def _(r):
            pltpu.sync_copy(acc_sh.at[pl.ds(row0 + r, ZBLK)],
                            out_hbm.at[cid, pl.ds(row0 + r, ZBLK)])

    return deg_kernel(dst_p)


def _make_agg_call(D):
    @functools.partial(
        pl.kernel,
        out_type=jax.ShapeDtypeStruct((NC, NP, D), _F32),
        mesh=_sc_mesh(),
        scratch_types=[
            pltpu.VMEM((max(N0, N1) // 2, CHUNK), jnp.int32),
            pltpu.VMEM((max(N0, N1) // 2, CHUNK), jnp.int32),
            pltpu.VMEM((CHUNK, D), _F32),
            pltpu.VMEM((CHUNK, D), _F32),
            pltpu.VMEM_SHARED((NP, D), _F32),
            pltpu.SemaphoreType.DMA,
            pltpu.SemaphoreType.DMA,
            pltpu.SemaphoreType.DMA,
            pltpu.SemaphoreType.DMA,
            pltpu.SemaphoreType.DMA,
        ],
    )
    def agg_kernel(hs_hbm, src_hbm, dst_hbm, out_hbm,
                   idx_src, idx_dst, rows_a, rows_b, acc_sh,
                   gsem_a, gsem_b, ssem_a, ssem_b, wsem):
        cid = lax.axis_index("c")
        sid = lax.axis_index("s")

        @pl.loop(0, CHUNK)
        def _(i):
            @pl.loop(0, D, step=LANES)
            def _(j):
                rows_a[i, pl.ds(j, LANES)] = jnp.zeros((LANES,), _F32)

        row0 = sid * ROWS_PER_TILE

        @pl.loop(0, ROWS_PER_TILE, step=CHUNK)
        def _(r):
            pltpu.sync_copy(rows_a, acc_sh.at[pl.ds(row0 + r, CHUNK)])

        plsc.subcore_barrier()

        def g_start(row, buf, sem):
            pltpu.async_copy(hs_hbm.at[idx_src.at[row]], buf, sem)

        def g_wait(row, buf, sem):
            pltpu.make_async_copy(hs_hbm.at[idx_src.at[row]], buf, sem).wait()

        def s_start(row, buf, sem):
            pltpu.async_copy(buf, acc_sh.at[idx_dst.at[row]], sem, add=True)

        def s_wait(row, buf, sem):
            pltpu.make_async_copy(buf, acc_sh.at[idx_dst.at[row]], sem).wait()

        def run_half(base, half):
            pltpu.sync_copy(src_hbm.at[pl.ds(base, half)],
                            idx_src.at[pl.ds(0, half)])
            pltpu.sync_copy(dst_hbm.at[pl.ds(base, half)],
                            idx_dst.at[pl.ds(0, half)])
            g_start(0, rows_a, gsem_a)
            g_start(1, rows_b, gsem_b)

            @pl.loop(0, half - 2, step=2)
            def _(p):
                g_wait(p, rows_a, gsem_a)
                s_start(p, rows_a, ssem_a)
                g_wait(p + 1, rows_b, gsem_b)
                s_start(p + 1, rows_b, ssem_b)
                s_wait(p, rows_a, ssem_a)
                g_start(p + 2, rows_a, gsem_a)
                s_wait(p + 1, rows_b, ssem_b)
                g_start(p + 3, rows_b, gsem_b)

            g_wait(half - 2, rows_a, gsem_a)
            s_start(half - 2, rows_a, ssem_a)
            g_wait(half - 1, rows_b, gsem_b)
            s_start(half - 1, rows_b, ssem_b)
            s_wait(half - 2, rows_a, ssem_a)
            s_wait(half - 1, rows_b, ssem_b)

        @pl.when(cid == 0)
        def _():
            base = sid * N0
            run_half(base, N0 // 2)
            run_half(base + N0 // 2, N0 // 2)

        @pl.when(cid == 1)
        def _():
            base = NS * N0 + sid * N1
            run_half(base, N1 // 2)
            run_half(base + N1 // 2, N1 // 2)

        plsc.subcore_barrier()

        @pl.loop(0, ROWS_PER_TILE, step=ZBLK)
        def _(r):
            pltpu.async_copy(acc_sh.at[pl.ds(row0 + r, ZBLK)],
                             out_hbm.at[cid, pl.ds(row0 + r, ZBLK)], wsem)

        @pl.loop(0, ROWS_PER_TILE, step=ZBLK)
        def _(r):
            pltpu.make_async_copy(acc_sh.at[pl.ds(row0 + r, ZBLK)],
                                  out_hbm.at[cid, pl.ds(row0 + r, ZBLK)],
                                  wsem).wait()

    return agg_kernel


def _prep_call(degp):
    BR = 1024

    def body(p_ref, dinv_ref, sn_ref):
        deg = p_ref[0, :, :1] + p_ref[1, :, :1] + 1.0
        dinv_ref[...] = lax.rsqrt(deg)
        sn_ref[...] = 1.0 / deg

    return pl.pallas_call(
        body,
        grid=(NP // BR,),
        in_specs=[pl.BlockSpec((NC, BR, 128), lambda i: (0, i, 0))],
        out_specs=[pl.BlockSpec((BR, 1), lambda i: (i, 0)),
                   pl.BlockSpec((BR, 1), lambda i: (i, 0))],
        out_shape=[jax.ShapeDtypeStruct((NP, 1), _F32)] * 2,
    )(degp)


def _l0_call(xp, W0, b0, dinv):
    BR = 512

    def body(x_ref, w_ref, b_ref, dinv_ref, h_ref, hs_ref):
        h = jnp.dot(x_ref[...], w_ref[...],
                    preferred_element_type=_F32, precision=_PREC) + b_ref[...]
        h_ref[...] = h
        hs_ref[...] = h * dinv_ref[...]

    return pl.pallas_call(
        body,
        grid=(NP // BR,),
        in_specs=[pl.BlockSpec((BR, D_IN), lambda i: (i, 0)),
                  pl.BlockSpec((D_IN, H), lambda i: (0, 0)),
                  pl.BlockSpec((1, H), lambda i: (0, 0)),
                  pl.BlockSpec((BR, 1), lambda i: (i, 0))],
        out_specs=[pl.BlockSpec((BR, H), lambda i: (i, 0))] * 2,
        out_shape=[jax.ShapeDtypeStruct((NP, H), _F32)] * 2,
    )(xp, W0, b0.reshape(1, H), dinv)


def _layer_call(aggp, h_prev, dinv, sn, W, b):
    BR = 512

    def body(p_ref, h_ref, dinv_ref, sn_ref, w_ref, b_ref,
             post_ref, hl_ref, hs_ref):
        dv = dinv_ref[...]
        agg = p_ref[0] + p_ref[1]
        post = jnp.maximum(dv * agg + h_ref[...] * sn_ref[...], 0.0)
        post_ref[...] = post
        hl = jnp.dot(post, w_ref[...],
                     preferred_element_type=_F32, precision=_PREC) + b_ref[...]
        hl_ref[...] = hl
        hs_ref[...] = hl * dv

    return pl.pallas_call(
        body,
        grid=(NP // BR,),
        in_specs=[pl.BlockSpec((NC, BR, H), lambda i: (0, i, 0)),
                  pl.BlockSpec((BR, H), lambda i: (i, 0)),
                  pl.BlockSpec((BR, 1), lambda i: (i, 0)),
                  pl.BlockSpec((BR, 1), lambda i: (i, 0)),
                  pl.BlockSpec((H, H), lambda i: (0, 0)),
                  pl.BlockSpec((1, H), lambda i: (0, 0))],
        out_specs=[pl.BlockSpec((BR, H), lambda i: (i, 0))] * 3,
        out_shape=[jax.ShapeDtypeStruct((NP, H), _F32)] * 3,
    )(aggp, h_prev, dinv, sn, W, b.reshape(1, H))


def _lstm_call(p1, p2, p3, agg4p, hl4, dinv, sn,
               Wxf, Whf, bgf, Wxb, Whb, bgb, fcf, fcb, W4, b4):
    BR = 256

    def body(p1_ref, p2_ref, p3_ref, a4_ref, hl4_ref, dinv_ref, sn_ref,
             wxf_ref, whf_ref, bf_ref, wxb_ref, whb_ref, bb_ref,
             fcf_ref, fcb_ref, w4_ref, b4_ref, h5_ref, hs5_ref):
        dv = dinv_ref[...]
        post4 = jnp.maximum(
            dv * (a4_ref[0] + a4_ref[1]) + hl4_ref[...] * sn_ref[...], 0.0)
        posts = [p1_ref[...], p2_ref[...], p3_ref[...], post4]

        def run(seq, wx, wh, bg):
            h = jnp.zeros((BR, UP), _F32)
            c = jnp.zeros((BR, UP), _F32)
            outs = []
            for xt in seq:
                za = [jnp.dot(xt, wx[g], preferred_element_type=_F32,
                              precision=_PREC)
                      + jnp.dot(h, wh[g], preferred_element_type=_F32,
                                precision=_PREC)
                      + bg[g] for g in range(4)]
                zi, zj, zf, zo = za
                c = (jax.nn.sigmoid(zf + 1.0) * c
                     + jax.nn.sigmoid(zi) * jnp.tanh(zj))
                h = jax.nn.sigmoid(zo) * jnp.tanh(c)
                outs.append(h)
            return outs

        fw = run(posts, wxf_ref[...], whf_ref[...], bf_ref[...])
        bw = run(posts[::-1], wxb_ref[...], whb_ref[...], bb_ref[...])[::-1]
        fcf_w = fcf_ref[...]
        fcb_w = fcb_ref[...]
        imps = []
        for t in range(4):
            it = (jnp.maximum(jnp.sum(fw[t] * fcf_w[t], axis=1,
                                      keepdims=True), 0.0)
                  + jnp.maximum(jnp.sum(bw[t] * fcb_w[t], axis=1,
                                        keepdims=True), 0.0))
            imps.append(it)
        m = jnp.maximum(jnp.maximum(imps[0], imps[1]),
                        jnp.maximum(imps[2], imps[3]))
        es = [jnp.exp(v - m) for v in imps]
        s = es[0] + es[1] + es[2] + es[3]
        summed = (es[0] / s) * posts[0]
        for t in range(1, 4):
            summed = summed + (es[t] / s) * posts[t]
        h5 = jnp.dot(summed, w4_ref[...],
                     preferred_element_type=_F32, precision=_PREC) + b4_ref[...]
        h5_ref[...] = h5
        hs5_ref[...] = h5 * dv

    nb = pl.BlockSpec((BR, H), lambda i: (i, 0))
    n1 = pl.BlockSpec((BR, 1), lambda i: (i, 0))
    full3 = lambda a: pl.BlockSpec(a.shape, lambda i: (0,) * a.ndim)
    return pl.pallas_call(
        body,
        grid=(NP // BR,),
        in_specs=[nb, nb, nb,
                  pl.BlockSpec((NC, BR, H), lambda i: (0, i, 0)),
                  nb, n1, n1,
                  full3(Wxf), full3(Whf), full3(bgf),
                  full3(Wxb), full3(Whb), full3(bgb),
                  full3(fcf), full3(fcb), full3(W4),
                  pl.BlockSpec((1, O_PAD), lambda i: (0, 0))],
        out_specs=[pl.BlockSpec((BR, O_PAD), lambda i: (i, 0))] * 2,
        out_shape=[jax.ShapeDtypeStruct((NP, O_PAD), _F32)] * 2,
    )(p1, p2, p3, agg4p, hl4, dinv, sn,
      Wxf, Whf, bgf, Wxb, Whb, bgb, fcf, fcb, W4, b4.reshape(1, O_PAD))


def _final_call(agg5p, h5, dinv, sn):
    BR = 1024

    def body(p_ref, h_ref, dinv_ref, sn_ref, o_ref):
        o_ref[...] = (dinv_ref[...] * (p_ref[0] + p_ref[1])
                      + h_ref[...] * sn_ref[...])

    return pl.pallas_call(
        body,
        grid=(NP // BR,),
        in_specs=[pl.BlockSpec((NC, BR, O_PAD), lambda i: (0, i, 0)),
                  pl.BlockSpec((BR, O_PAD), lambda i: (i, 0)),
                  pl.BlockSpec((BR, 1), lambda i: (i, 0)),
                  pl.BlockSpec((BR, 1), lambda i: (i, 0))],
        out_specs=pl.BlockSpec((BR, O_PAD), lambda i: (i, 0)),
        out_shape=jax.ShapeDtypeStruct((NP, O_PAD), _F32),
    )(agg5p, h5, dinv, sn)


def _pad_gates(Wx, Wh, b):
    Wxg = jnp.stack([jnp.pad(Wx[:, g * U:(g + 1) * U], ((0, 0), (0, UP - U)))
                     for g in range(4)])
    Whg = jnp.stack([jnp.pad(Wh[:, g * U:(g + 1) * U],
                             ((0, UP - U), (0, UP - U)))
                     for g in range(4)])
    bg = jnp.stack([jnp.pad(b[g * U:(g + 1) * U], (0, UP - U))
                    for g in range(4)])
    return Wxg, Whg, bg


def kernel(x, edge_index, W0, b0, W1, b1, W2, b2, W3, b3,
           Wx_fw, Wh_fw, bl_fw, Wx_bw, Wh_bw, bl_bw, fc_fw, fc_bw, W4, b4):
    src, dst = edge_index[0], edge_index[1]
    src_p = jnp.concatenate(
        [src, jnp.zeros((E_PAD - E_EDGES,), jnp.int32)]).reshape(
            TOT_CH, CHUNK)
    dst_p = jnp.concatenate(
        [dst, jnp.full((E_PAD - E_EDGES,), N_NODES, jnp.int32)]).reshape(
            TOT_CH, CHUNK)
    xp = jnp.concatenate([x, jnp.zeros((NP - N_NODES, D_IN), x.dtype)])

    degp = _deg_call(dst_p.reshape(NW, NCH, CHUNK))
    dinv, sn = _prep_call(degp)

    agg_h = _make_agg_call(H)
    hl, hs = _l0_call(xp, W0, b0, dinv)
    posts = []
    for Wn, bn in ((W1, b1), (W2, b2), (W3, b3)):
        aggp = agg_h(hs, src_p, dst_p)
        post, hl, hs = _layer_call(aggp, hl, dinv, sn, Wn, bn)
        posts.append(post)
    agg4p = agg_h(hs, src_p, dst_p)

    Wxf, Whf, bgf = _pad_gates(Wx_fw, Wh_fw, bl_fw)
    Wxb, Whb, bgb = _pad_gates(Wx_bw, Wh_bw, bl_bw)
    fcf = jnp.pad(fc_fw[:, :, 0], ((0, 0), (0, UP - U)))
    fcb = jnp.pad(fc_bw[:, :, 0], ((0, 0), (0, UP - U)))

    W4p = jnp.pad(W4, ((0, 0), (0, O_PAD - O_DIM)))
    b4p = jnp.pad(b4, (0, O_PAD - O_DIM))
    h5, hs5 = _lstm_call(posts[0], posts[1], posts[2], agg4p, hl, dinv, sn,
                         Wxf, Whf, bgf, Wxb, Whb, bgb, fcf, fcb, W4p, b4p)

    agg5p = _make_agg_call(O_PAD)(hs5, src_p, dst_p)
    out = _final_call(agg5p, h5, dinv, sn)
    return out[:N_NODES, :O_DIM]


# speedup vs baseline: 1.0390x; 1.0390x over previous
import functools

import jax
import jax.numpy as jnp
from jax import lax
from jax.experimental import pallas as pl
from jax.experimental.pallas import tpu as pltpu
from jax.experimental.pallas import tpu_sc as plsc

N_NODES = 10000
NP = 10240
E_EDGES = 320000
D_IN = 128
H = 128
O_DIM = 64
O_PAD = 128
U = 50
UP = 64
NC, NS, LANES = 2, 16, 16
NW = NC * NS
CHUNK = 128
NCH = 80
PER_W = NCH * CHUNK
E_PAD = PER_W * NW
TOT_CH = E_PAD // CHUNK
N0 = 128
N1 = 32
assert NS * (N0 + N1) == TOT_CH and N0 % 16 == 0 and N1 % 16 == 0
ROWS_PER_TILE = NP // NS
ZBLK = 128

_PREC = jax.lax.Precision.HIGHEST
_F32 = jnp.float32


def _sc_mesh():
    return plsc.VectorSubcoreMesh(core_axis_name="c", subcore_axis_name="s",
                                  num_cores=NC, num_subcores=NS)


def _deg_call(dst_p):
    @functools.partial(
        pl.kernel,
        out_type=jax.ShapeDtypeStruct((NC, NP, 128), _F32),
        mesh=_sc_mesh(),
        scratch_types=[
            pltpu.VMEM((max(N0, N1), CHUNK), jnp.int32),
            pltpu.VMEM((CHUNK, 128), _F32),
            pltpu.VMEM((ZBLK, 128), _F32),
            pltpu.VMEM_SHARED((NP, 128), _F32),
            pltpu.SemaphoreType.DMA,
        ],
    )
    def deg_kernel(dst_hbm, out_hbm, dst_all, ones_v, zero_v, acc_sh, sem):
        cid = lax.axis_index("c")
        sid = lax.axis_index("s")

        @pl.loop(0, CHUNK)
        def _(i):
            @pl.loop(0, 128, step=LANES)
            def _(j):
                ones_v[i, pl.ds(j, LANES)] = jnp.ones((LANES,), _F32)

        @pl.loop(0, ZBLK)
        def _(i):
            @pl.loop(0, 128, step=LANES)
            def _(j):
                zero_v[i, pl.ds(j, LANES)] = jnp.zeros((LANES,), _F32)

        row0 = sid * ROWS_PER_TILE

        @pl.loop(0, ROWS_PER_TILE, step=ZBLK)
        def _(r):
            pltpu.sync_copy(zero_v, acc_sh.at[pl.ds(row0 + r, ZBLK)])

        plsc.subcore_barrier()

        def scatter_ones(nch, base):
            pltpu.sync_copy(dst_hbm.at[pl.ds(base, nch)],
                            dst_all.at[pl.ds(0, nch)])

            @pl.loop(0, nch, step=8)
            def _(c0):
                for k in range(8):
                    pltpu.async_copy(ones_v, acc_sh.at[dst_all.at[c0 + k]],
                                     sem, add=True)
                for k in range(8):
                    pltpu.make_async_copy(ones_v,
                                          acc_sh.at[dst_all.at[c0 + k]],
                                          sem).wait()

        @pl.when(cid == 0)
        def _():
            scatter_ones(N0, sid * N0)

        @pl.when(cid == 1)
        def _():
            scatter_ones(N1, NS * N0 + sid * N1)

        plsc.subcore_barrier()

        @pl.loop(0, ROWS_PER_TILE, step=ZBLK)
        def _(r):
            pltpu.sync_copy(acc_sh.at[pl.ds(row0 + r, ZBLK)],
                            out_hbm.at[cid, pl.ds(row0 + r, ZBLK)])

    return deg_kernel(dst_p)


def _make_agg_call(D):
    @functools.partial(
        pl.kernel,
        out_type=jax.ShapeDtypeStruct((NC, NP, D), _F32),
        mesh=_sc_mesh(),
        scratch_types=[
            pltpu.VMEM((max(N0, N1) // 2, CHUNK), jnp.int32),
            pltpu.VMEM((max(N0, N1) // 2, CHUNK), jnp.int32),
            pltpu.VMEM((CHUNK, D), _F32),
            pltpu.VMEM((CHUNK, D), _F32),
            pltpu.VMEM_SHARED((NP, D), _F32),
            pltpu.SemaphoreType.DMA,
            pltpu.SemaphoreType.DMA,
            pltpu.SemaphoreType.DMA,
            pltpu.SemaphoreType.DMA,
            pltpu.SemaphoreType.DMA,
        ],
    )
    def agg_kernel(hs_hbm, src_hbm, dst_hbm, out_hbm,
                   idx_src, idx_dst, rows_a, rows_b, acc_sh,
                   gsem_a, gsem_b, ssem_a, ssem_b, wsem):
        cid = lax.axis_index("c")
        sid = lax.axis_index("s")

        @pl.loop(0, CHUNK)
        def _(i):
            @pl.loop(0, D, step=LANES)
            def _(j):
                rows_a[i, pl.ds(j, LANES)] = jnp.zeros((LANES,), _F32)

        row0 = sid * ROWS_PER_TILE

        @pl.loop(0, ROWS_PER_TILE, step=CHUNK)
        def _(r):
            pltpu.sync_copy(rows_a, acc_sh.at[pl.ds(row0 + r, CHUNK)])

        plsc.subcore_barrier()

        def g_start(row, buf, sem):
            pltpu.async_copy(hs_hbm.at[idx_src.at[row]], buf, sem)

        def g_wait(row, buf, sem):
            pltpu.make_async_copy(hs_hbm.at[idx_src.at[row]], buf, sem).wait()

        def s_start(row, buf, sem):
            pltpu.async_copy(buf, acc_sh.at[idx_dst.at[row]], sem, add=True)

        def s_wait(row, buf, sem):
            pltpu.make_async_copy(buf, acc_sh.at[idx_dst.at[row]], sem).wait()

        def run_half(base, half):
            pltpu.sync_copy(src_hbm.at[pl.ds(base, half)],
                            idx_src.at[pl.ds(0, half)])
            pltpu.sync_copy(dst_hbm.at[pl.ds(base, half)],
                            idx_dst.at[pl.ds(0, half)])
            g_start(0, rows_a, gsem_a)
            g_start(1, rows_b, gsem_b)

            @pl.loop(0, half - 2, step=2)
            def _(p):
                g_wait(p, rows_a, gsem_a)
                s_start(p, rows_a, ssem_a)
                g_wait(p + 1, rows_b, gsem_b)
                s_start(p + 1, rows_b, ssem_b)
                s_wait(p, rows_a, ssem_a)
                g_start(p + 2, rows_a, gsem_a)
                s_wait(p + 1, rows_b, ssem_b)
                g_start(p + 3, rows_b, gsem_b)

            g_wait(half - 2, rows_a, gsem_a)
            s_start(half - 2, rows_a, ssem_a)
            g_wait(half - 1, rows_b, gsem_b)
            s_start(half - 1, rows_b, ssem_b)
            s_wait(half - 2, rows_a, ssem_a)
            s_wait(half - 1, rows_b, ssem_b)

        @pl.when(cid == 0)
        def _():
            base = sid * N0
            run_half(base, N0 // 2)
            run_half(base + N0 // 2, N0 // 2)

        @pl.when(cid == 1)
        def _():
            base = NS * N0 + sid * N1
            run_half(base, N1 // 2)
            run_half(base + N1 // 2, N1 // 2)

        plsc.subcore_barrier()

        @pl.loop(0, ROWS_PER_TILE, step=ZBLK)
        def _(r):
            pltpu.async_copy(acc_sh.at[pl.ds(row0 + r, ZBLK)],
                             out_hbm.at[cid, pl.ds(row0 + r, ZBLK)], wsem)

        @pl.loop(0, ROWS_PER_TILE, step=ZBLK)
        def _(r):
            pltpu.make_async_copy(acc_sh.at[pl.ds(row0 + r, ZBLK)],
                                  out_hbm.at[cid, pl.ds(row0 + r, ZBLK)],
                                  wsem).wait()

    return agg_kernel


def _prep_call(degp):
    BR = 1024

    def body(p_ref, dinv_ref, sn_ref):
        deg = p_ref[0, :, :1] + p_ref[1, :, :1] + 1.0
        dinv_ref[...] = lax.rsqrt(deg)
        sn_ref[...] = 1.0 / deg

    return pl.pallas_call(
        body,
        grid=(NP // BR,),
        in_specs=[pl.BlockSpec((NC, BR, 128), lambda i: (0, i, 0))],
        out_specs=[pl.BlockSpec((BR, 1), lambda i: (i, 0)),
                   pl.BlockSpec((BR, 1), lambda i: (i, 0))],
        out_shape=[jax.ShapeDtypeStruct((NP, 1), _F32)] * 2,
    )(degp)


def _l0_matmul(xp, W0, b0):
    BR = 512

    def body(x_ref, w_ref, b_ref, h_ref):
        h_ref[...] = jnp.dot(x_ref[...], w_ref[...],
                             preferred_element_type=_F32,
                             precision=_PREC) + b_ref[...]

    return pl.pallas_call(
        body,
        grid=(NP // BR,),
        in_specs=[pl.BlockSpec((BR, D_IN), lambda i: (i, 0)),
                  pl.BlockSpec((D_IN, H), lambda i: (0, 0)),
                  pl.BlockSpec((1, H), lambda i: (0, 0))],
        out_specs=pl.BlockSpec((BR, H), lambda i: (i, 0)),
        out_shape=jax.ShapeDtypeStruct((NP, H), _F32),
    )(xp, W0, b0.reshape(1, H))


def _scale_call(h, dinv):
    BR = 1024

    def body(h_ref, dinv_ref, hs_ref):
        hs_ref[...] = h_ref[...] * dinv_ref[...]

    return pl.pallas_call(
        body,
        grid=(NP // BR,),
        in_specs=[pl.BlockSpec((BR, H), lambda i: (i, 0)),
                  pl.BlockSpec((BR, 1), lambda i: (i, 0))],
        out_specs=pl.BlockSpec((BR, H), lambda i: (i, 0)),
        out_shape=jax.ShapeDtypeStruct((NP, H), _F32),
    )(h, dinv)


def _layer_call(aggp, h_prev, dinv, sn, W, b):
    BR = 512

    def body(p_ref, h_ref, dinv_ref, sn_ref, w_ref, b_ref,
             post_ref, hl_ref, hs_ref):
        dv = dinv_ref[...]
        agg = p_ref[0] + p_ref[1]
        post = jnp.maximum(dv * agg + h_ref[...] * sn_ref[...], 0.0)
        post_ref[...] = post
        hl = jnp.dot(post, w_ref[...],
                     preferred_element_type=_F32, precision=_PREC) + b_ref[...]
        hl_ref[...] = hl
        hs_ref[...] = hl * dv

    return pl.pallas_call(
        body,
        grid=(NP // BR,),
        in_specs=[pl.BlockSpec((NC, BR, H), lambda i: (0, i, 0)),
                  pl.BlockSpec((BR, H), lambda i: (i, 0)),
                  pl.BlockSpec((BR, 1), lambda i: (i, 0)),
                  pl.BlockSpec((BR, 1), lambda i: (i, 0)),
                  pl.BlockSpec((H, H), lambda i: (0, 0)),
                  pl.BlockSpec((1, H), lambda i: (0, 0))],
        out_specs=[pl.BlockSpec((BR, H), lambda i: (i, 0))] * 3,
        out_shape=[jax.ShapeDtypeStruct((NP, H), _F32)] * 3,
    )(aggp, h_prev, dinv, sn, W, b.reshape(1, H))


def _lstm_call(p1, p2, p3, agg4p, hl4, dinv, sn,
               Wxf, Whf, bgf, Wxb, Whb, bgb, fcf, fcb, W4, b4):
    BR = 256

    def body(p1_ref, p2_ref, p3_ref, a4_ref, hl4_ref, dinv_ref, sn_ref,
             wxf_ref, whf_ref, bf_ref, wxb_ref, whb_ref, bb_ref,
             fcf_ref, fcb_ref, w4_ref, b4_ref, h5_ref, hs5_ref):
        dv = dinv_ref[...]
        post4 = jnp.maximum(
            dv * (a4_ref[0] + a4_ref[1]) + hl4_ref[...] * sn_ref[...], 0.0)
        posts = [p1_ref[...], p2_ref[...], p3_ref[...], post4]

        def run(seq, wx, wh, bg):
            h = jnp.zeros((BR, UP), _F32)
            c = jnp.zeros((BR, UP), _F32)
            outs = []
            for xt in seq:
                za = [jnp.dot(xt, wx[g], preferred_element_type=_F32,
                              precision=_PREC)
                      + jnp.dot(h, wh[g], preferred_element_type=_F32,
                                precision=_PREC)
                      + bg[g] for g in range(4)]
                zi, zj, zf, zo = za
                c = (jax.nn.sigmoid(zf + 1.0) * c
                     + jax.nn.sigmoid(zi) * jnp.tanh(zj))
                h = jax.nn.sigmoid(zo) * jnp.tanh(c)
                outs.append(h)
            return outs

        fw = run(posts, wxf_ref[...], whf_ref[...], bf_ref[...])
        bw = run(posts[::-1], wxb_ref[...], whb_ref[...], bb_ref[...])[::-1]
        fcf_w = fcf_ref[...]
        fcb_w = fcb_ref[...]
        imps = []
        for t in range(4):
            it = (jnp.maximum(jnp.sum(fw[t] * fcf_w[t], axis=1,
                                      keepdims=True), 0.0)
                  + jnp.maximum(jnp.sum(bw[t] * fcb_w[t], axis=1,
                                        keepdims=True), 0.0))
            imps.append(it)
        m = jnp.maximum(jnp.maximum(imps[0], imps[1]),
                        jnp.maximum(imps[2], imps[3]))
        es = [jnp.exp(v - m) for v in imps]
        s = es[0] + es[1] + es[2] + es[3]
        summed = (es[0] / s) * posts[0]
        for t in range(1, 4):
            summed = summed + (es[t] / s) * posts[t]
        h5 = jnp.dot(summed, w4_ref[...],
                     preferred_element_type=_F32, precision=_PREC) + b4_ref[...]
        h5_ref[...] = h5
        hs5_ref[...] = h5 * dv

    nb = pl.BlockSpec((BR, H), lambda i: (i, 0))
    n1 = pl.BlockSpec((BR, 1), lambda i: (i, 0))
    full3 = lambda a: pl.BlockSpec(a.shape, lambda i: (0,) * a.ndim)
    return pl.pallas_call(
        body,
        grid=(NP // BR,),
        in_specs=[nb, nb, nb,
                  pl.BlockSpec((NC, BR, H), lambda i: (0, i, 0)),
                  nb, n1, n1,
                  full3(Wxf), full3(Whf), full3(bgf),
                  full3(Wxb), full3(Whb), full3(bgb),
                  full3(fcf), full3(fcb), full3(W4),
                  pl.BlockSpec((1, O_PAD), lambda i: (0, 0))],
        out_specs=[pl.BlockSpec((BR, O_PAD), lambda i: (i, 0))] * 2,
        out_shape=[jax.ShapeDtypeStruct((NP, O_PAD), _F32)] * 2,
    )(p1, p2, p3, agg4p, hl4, dinv, sn,
      Wxf, Whf, bgf, Wxb, Whb, bgb, fcf, fcb, W4, b4.reshape(1, O_PAD))


def _final_call(agg5p, h5, dinv, sn):
    BR = 1024

    def body(p_ref, h_ref, dinv_ref, sn_ref, o_ref):
        o_ref[...] = (dinv_ref[...] * (p_ref[0] + p_ref[1])
                      + h_ref[...] * sn_ref[...])

    return pl.pallas_call(
        body,
        grid=(NP // BR,),
        in_specs=[pl.BlockSpec((NC, BR, O_PAD), lambda i: (0, i, 0)),
                  pl.BlockSpec((BR, O_PAD), lambda i: (i, 0)),
                  pl.BlockSpec((BR, 1), lambda i: (i, 0)),
                  pl.BlockSpec((BR, 1), lambda i: (i, 0))],
        out_specs=pl.BlockSpec((BR, O_PAD), lambda i: (i, 0)),
        out_shape=jax.ShapeDtypeStruct((NP, O_PAD), _F32),
    )(agg5p, h5, dinv, sn)


def _pad_gates(Wx, Wh, b):
    Wxg = jnp.stack([jnp.pad(Wx[:, g * U:(g + 1) * U], ((0, 0), (0, UP - U)))
                     for g in range(4)])
    Whg = jnp.stack([jnp.pad(Wh[:, g * U:(g + 1) * U],
                             ((0, UP - U), (0, UP - U)))
                     for g in range(4)])
    bg = jnp.stack([jnp.pad(b[g * U:(g + 1) * U], (0, UP - U))
                    for g in range(4)])
    return Wxg, Whg, bg


def kernel(x, edge_index, W0, b0, W1, b1, W2, b2, W3, b3,
           Wx_fw, Wh_fw, bl_fw, Wx_bw, Wh_bw, bl_bw, fc_fw, fc_bw, W4, b4):
    src, dst = edge_index[0], edge_index[1]
    src_p = jnp.concatenate(
        [src, jnp.zeros((E_PAD - E_EDGES,), jnp.int32)]).reshape(
            TOT_CH, CHUNK)
    dst_p = jnp.concatenate(
        [dst, jnp.full((E_PAD - E_EDGES,), N_NODES, jnp.int32)]).reshape(
            TOT_CH, CHUNK)
    xp = jnp.concatenate([x, jnp.zeros((NP - N_NODES, D_IN), x.dtype)])

    degp = _deg_call(dst_p)
    dinv, sn = _prep_call(degp)

    agg_h = _make_agg_call(H)
    hl = _l0_matmul(xp, W0, b0)
    hs = _scale_call(hl, dinv)
    posts = []
    for Wn, bn in ((W1, b1), (W2, b2), (W3, b3)):
        aggp = agg_h(hs, src_p, dst_p)
        post, hl, hs = _layer_call(aggp, hl, dinv, sn, Wn, bn)
        posts.append(post)
    agg4p = agg_h(hs, src_p, dst_p)

    Wxf, Whf, bgf = _pad_gates(Wx_fw, Wh_fw, bl_fw)
    Wxb, Whb, bgb = _pad_gates(Wx_bw, Wh_bw, bl_bw)
    fcf = jnp.pad(fc_fw[:, :, 0], ((0, 0), (0, UP - U)))
    fcb = jnp.pad(fc_bw[:, :, 0], ((0, 0), (0, UP - U)))

    W4p = jnp.pad(W4, ((0, 0), (0, O_PAD - O_DIM)))
    b4p = jnp.pad(b4, (0, O_PAD - O_DIM))
    h5, hs5 = _lstm_call(posts[0], posts[1], posts[2], agg4p, hl, dinv, sn,
                         Wxf, Whf, bgf, Wxb, Whb, bgb, fcf, fcb, W4p, b4p)

    agg5p = _make_agg_call(O_PAD)(hs5, src_p, dst_p)
    out = _final_call(agg5p, h5, dinv, sn)
    return out[:N_NODES, :O_DIM]


# speedup vs baseline: 1.0526x; 1.0131x over previous
import functools

import jax
import jax.numpy as jnp
from jax import lax
from jax.experimental import pallas as pl
from jax.experimental.pallas import tpu as pltpu
from jax.experimental.pallas import tpu_sc as plsc

N_NODES = 10000
NP = 10240
E_EDGES = 320000
D_IN = 128
H = 128
O_DIM = 64
O_PAD = 128
U = 50
UP = 64
NC, NS, LANES = 2, 16, 16
NW = NC * NS
CHUNK = 128
NCH = 80
PER_W = NCH * CHUNK
E_PAD = PER_W * NW
TOT_CH = E_PAD // CHUNK
N0 = 128
N1 = 32
assert NS * (N0 + N1) == TOT_CH and N0 % 16 == 0 and N1 % 16 == 0
ROWS_PER_TILE = NP // NS
ZBLK = 128

_PREC = jax.lax.Precision.HIGHEST
_F32 = jnp.float32


def _sc_mesh():
    return plsc.VectorSubcoreMesh(core_axis_name="c", subcore_axis_name="s",
                                  num_cores=NC, num_subcores=NS)


def _deg_call(dst_p):
    @functools.partial(
        pl.kernel,
        out_type=jax.ShapeDtypeStruct((NC, NP, 128), _F32),
        mesh=_sc_mesh(),
        scratch_types=[
            pltpu.VMEM((NCH, CHUNK), jnp.int32),
            pltpu.VMEM((CHUNK, 128), _F32),
            pltpu.VMEM((ZBLK, 128), _F32),
            pltpu.VMEM_SHARED((NP, 128), _F32),
            pltpu.SemaphoreType.DMA,
        ],
    )
    def deg_kernel(dst_hbm, out_hbm, dst_all, ones_v, zero_v, acc_sh, sem):
        cid = lax.axis_index("c")
        sid = lax.axis_index("s")

        @pl.loop(0, CHUNK)
        def _(i):
            @pl.loop(0, 128, step=LANES)
            def _(j):
                ones_v[i, pl.ds(j, LANES)] = jnp.ones((LANES,), _F32)

        @pl.loop(0, ZBLK)
        def _(i):
            @pl.loop(0, 128, step=LANES)
            def _(j):
                zero_v[i, pl.ds(j, LANES)] = jnp.zeros((LANES,), _F32)

        row0 = sid * ROWS_PER_TILE

        @pl.loop(0, ROWS_PER_TILE, step=ZBLK)
        def _(r):
            pltpu.sync_copy(zero_v, acc_sh.at[pl.ds(row0 + r, ZBLK)])

        plsc.subcore_barrier()

        def scatter_ones(nch, base):
            pltpu.sync_copy(dst_hbm.at[pl.ds(base, nch)],
                            dst_all.at[pl.ds(0, nch)])

            @pl.loop(0, nch, step=8)
            def _(c0):
                for k in range(8):
                    pltpu.async_copy(ones_v, acc_sh.at[dst_all.at[c0 + k]],
                                     sem, add=True)
                for k in range(8):
                    pltpu.make_async_copy(ones_v,
                                          acc_sh.at[dst_all.at[c0 + k]],
                                          sem).wait()

        tile = cid * NS + sid
        scatter_ones(NCH, tile * NCH)

        plsc.subcore_barrier()

        @pl.loop(0, ROWS_PER_TILE, step=ZBLK)
        def _(r):
            pltpu.sync_copy(acc_sh.at[pl.ds(row0 + r, ZBLK)],
                            out_hbm.at[cid, pl.ds(row0 + r, ZBLK)])

    return deg_kernel(dst_p)


def _make_agg_call(D):
    @functools.partial(
        pl.kernel,
        out_type=jax.ShapeDtypeStruct((NC, NP, D), _F32),
        mesh=_sc_mesh(),
        scratch_types=[
            pltpu.VMEM((max(N0, N1) // 2, CHUNK), jnp.int32),
            pltpu.VMEM((max(N0, N1) // 2, CHUNK), jnp.int32),
            pltpu.VMEM((CHUNK, D), _F32),
            pltpu.VMEM((CHUNK, D), _F32),
            pltpu.VMEM_SHARED((NP, D), _F32),
            pltpu.SemaphoreType.DMA,
            pltpu.SemaphoreType.DMA,
            pltpu.SemaphoreType.DMA,
            pltpu.SemaphoreType.DMA,
            pltpu.SemaphoreType.DMA,
        ],
    )
    def agg_kernel(hs_hbm, src_hbm, dst_hbm, out_hbm,
                   idx_src, idx_dst, rows_a, rows_b, acc_sh,
                   gsem_a, gsem_b, ssem_a, ssem_b, wsem):
        cid = lax.axis_index("c")
        sid = lax.axis_index("s")

        @pl.loop(0, CHUNK)
        def _(i):
            @pl.loop(0, D, step=LANES)
            def _(j):
                rows_a[i, pl.ds(j, LANES)] = jnp.zeros((LANES,), _F32)

        row0 = sid * ROWS_PER_TILE

        @pl.loop(0, ROWS_PER_TILE, step=CHUNK)
        def _(r):
            pltpu.sync_copy(rows_a, acc_sh.at[pl.ds(row0 + r, CHUNK)])

        plsc.subcore_barrier()

        def g_start(row, buf, sem):
            pltpu.async_copy(hs_hbm.at[idx_src.at[row]], buf, sem)

        def g_wait(row, buf, sem):
            pltpu.make_async_copy(hs_hbm.at[idx_src.at[row]], buf, sem).wait()

        def s_start(row, buf, sem):
            pltpu.async_copy(buf, acc_sh.at[idx_dst.at[row]], sem, add=True)

        def s_wait(row, buf, sem):
            pltpu.make_async_copy(buf, acc_sh.at[idx_dst.at[row]], sem).wait()

        def run_half(base, half):
            pltpu.sync_copy(src_hbm.at[pl.ds(base, half)],
                            idx_src.at[pl.ds(0, half)])
            pltpu.sync_copy(dst_hbm.at[pl.ds(base, half)],
                            idx_dst.at[pl.ds(0, half)])
            g_start(0, rows_a, gsem_a)
            g_start(1, rows_b, gsem_b)

            @pl.loop(0, half - 2, step=2)
            def _(p):
                g_wait(p, rows_a, gsem_a)
                s_start(p, rows_a, ssem_a)
                g_wait(p + 1, rows_b, gsem_b)
                s_start(p + 1, rows_b, ssem_b)
                s_wait(p, rows_a, ssem_a)
                g_start(p + 2, rows_a, gsem_a)
                s_wait(p + 1, rows_b, ssem_b)
                g_start(p + 3, rows_b, gsem_b)

            g_wait(half - 2, rows_a, gsem_a)
            s_start(half - 2, rows_a, ssem_a)
            g_wait(half - 1, rows_b, gsem_b)
            s_start(half - 1, rows_b, ssem_b)
            s_wait(half - 2, rows_a, ssem_a)
            s_wait(half - 1, rows_b, ssem_b)

        @pl.when(cid == 0)
        def _():
            base = sid * N0
            run_half(base, N0 // 2)
            run_half(base + N0 // 2, N0 // 2)

        @pl.when(cid == 1)
        def _():
            base = NS * N0 + sid * N1
            run_half(base, N1 // 2)
            run_half(base + N1 // 2, N1 // 2)

        plsc.subcore_barrier()

        @pl.loop(0, ROWS_PER_TILE, step=ZBLK)
        def _(r):
            pltpu.async_copy(acc_sh.at[pl.ds(row0 + r, ZBLK)],
                             out_hbm.at[cid, pl.ds(row0 + r, ZBLK)], wsem)

        @pl.loop(0, ROWS_PER_TILE, step=ZBLK)
        def _(r):
            pltpu.make_async_copy(acc_sh.at[pl.ds(row0 + r, ZBLK)],
                                  out_hbm.at[cid, pl.ds(row0 + r, ZBLK)],
                                  wsem).wait()

    return agg_kernel


def _prep_call(degp):
    BR = 1024

    def body(p_ref, dinv_ref, sn_ref):
        deg = p_ref[0, :, :1] + p_ref[1, :, :1] + 1.0
        dinv_ref[...] = lax.rsqrt(deg)
        sn_ref[...] = 1.0 / deg

    return pl.pallas_call(
        body,
        grid=(NP // BR,),
        in_specs=[pl.BlockSpec((NC, BR, 128), lambda i: (0, i, 0))],
        out_specs=[pl.BlockSpec((BR, 1), lambda i: (i, 0)),
                   pl.BlockSpec((BR, 1), lambda i: (i, 0))],
        out_shape=[jax.ShapeDtypeStruct((NP, 1), _F32)] * 2,
    )(degp)


def _l0_matmul(xp, W0, b0):
    BR = 512

    def body(x_ref, w_ref, b_ref, h_ref):
        h_ref[...] = jnp.dot(x_ref[...], w_ref[...],
                             preferred_element_type=_F32,
                             precision=_PREC) + b_ref[...]

    return pl.pallas_call(
        body,
        grid=(NP // BR,),
        in_specs=[pl.BlockSpec((BR, D_IN), lambda i: (i, 0)),
                  pl.BlockSpec((D_IN, H), lambda i: (0, 0)),
                  pl.BlockSpec((1, H), lambda i: (0, 0))],
        out_specs=pl.BlockSpec((BR, H), lambda i: (i, 0)),
        out_shape=jax.ShapeDtypeStruct((NP, H), _F32),
    )(xp, W0, b0.reshape(1, H))


def _scale_call(h, dinv):
    BR = 1024

    def body(h_ref, dinv_ref, hs_ref):
        hs_ref[...] = h_ref[...] * dinv_ref[...]

    return pl.pallas_call(
        body,
        grid=(NP // BR,),
        in_specs=[pl.BlockSpec((BR, H), lambda i: (i, 0)),
                  pl.BlockSpec((BR, 1), lambda i: (i, 0))],
        out_specs=pl.BlockSpec((BR, H), lambda i: (i, 0)),
        out_shape=jax.ShapeDtypeStruct((NP, H), _F32),
    )(h, dinv)


def _layer_call(aggp, h_prev, dinv, sn, W, b):
    BR = 512

    def body(p_ref, h_ref, dinv_ref, sn_ref, w_ref, b_ref,
             post_ref, hl_ref, hs_ref):
        dv = dinv_ref[...]
        agg = p_ref[0] + p_ref[1]
        post = jnp.maximum(dv * agg + h_ref[...] * sn_ref[...], 0.0)
        post_ref[...] = post
        hl = jnp.dot(post, w_ref[...],
                     preferred_element_type=_F32, precision=_PREC) + b_ref[...]
        hl_ref[...] = hl
        hs_ref[...] = hl * dv

    return pl.pallas_call(
        body,
        grid=(NP // BR,),
        in_specs=[pl.BlockSpec((NC, BR, H), lambda i: (0, i, 0)),
                  pl.BlockSpec((BR, H), lambda i: (i, 0)),
                  pl.BlockSpec((BR, 1), lambda i: (i, 0)),
                  pl.BlockSpec((BR, 1), lambda i: (i, 0)),
                  pl.BlockSpec((H, H), lambda i: (0, 0)),
                  pl.BlockSpec((1, H), lambda i: (0, 0))],
        out_specs=[pl.BlockSpec((BR, H), lambda i: (i, 0))] * 3,
        out_shape=[jax.ShapeDtypeStruct((NP, H), _F32)] * 3,
    )(aggp, h_prev, dinv, sn, W, b.reshape(1, H))


def _lstm_call(p1, p2, p3, agg4p, hl4, dinv, sn,
               Wxf, Whf, bgf, Wxb, Whb, bgb, fcf, fcb, W4, b4):
    BR = 256

    def body(p1_ref, p2_ref, p3_ref, a4_ref, hl4_ref, dinv_ref, sn_ref,
             wxf_ref, whf_ref, bf_ref, wxb_ref, whb_ref, bb_ref,
             fcf_ref, fcb_ref, w4_ref, b4_ref, h5_ref, hs5_ref):
        dv = dinv_ref[...]
        post4 = jnp.maximum(
            dv * (a4_ref[0] + a4_ref[1]) + hl4_ref[...] * sn_ref[...], 0.0)
        posts = [p1_ref[...], p2_ref[...], p3_ref[...], post4]

        def run(seq, wx, wh, bg):
            h = jnp.zeros((BR, UP), _F32)
            c = jnp.zeros((BR, UP), _F32)
            outs = []
            for xt in seq:
                za = [jnp.dot(xt, wx[g], preferred_element_type=_F32,
                              precision=_PREC)
                      + jnp.dot(h, wh[g], preferred_element_type=_F32,
                                precision=_PREC)
                      + bg[g] for g in range(4)]
                zi, zj, zf, zo = za
                c = (jax.nn.sigmoid(zf + 1.0) * c
                     + jax.nn.sigmoid(zi) * jnp.tanh(zj))
                h = jax.nn.sigmoid(zo) * jnp.tanh(c)
                outs.append(h)
            return outs

        fw = run(posts, wxf_ref[...], whf_ref[...], bf_ref[...])
        bw = run(posts[::-1], wxb_ref[...], whb_ref[...], bb_ref[...])[::-1]
        fcf_w = fcf_ref[...]
        fcb_w = fcb_ref[...]
        imps = []
        for t in range(4):
            it = (jnp.maximum(jnp.sum(fw[t] * fcf_w[t], axis=1,
                                      keepdims=True), 0.0)
                  + jnp.maximum(jnp.sum(bw[t] * fcb_w[t], axis=1,
                                        keepdims=True), 0.0))
            imps.append(it)
        m = jnp.maximum(jnp.maximum(imps[0], imps[1]),
                        jnp.maximum(imps[2], imps[3]))
        es = [jnp.exp(v - m) for v in imps]
        s = es[0] + es[1] + es[2] + es[3]
        summed = (es[0] / s) * posts[0]
        for t in range(1, 4):
            summed = summed + (es[t] / s) * posts[t]
        h5 = jnp.dot(summed, w4_ref[...],
                     preferred_element_type=_F32, precision=_PREC) + b4_ref[...]
        h5_ref[...] = h5
        hs5_ref[...] = h5 * dv

    nb = pl.BlockSpec((BR, H), lambda i: (i, 0))
    n1 = pl.BlockSpec((BR, 1), lambda i: (i, 0))
    full3 = lambda a: pl.BlockSpec(a.shape, lambda i: (0,) * a.ndim)
    return pl.pallas_call(
        body,
        grid=(NP // BR,),
        in_specs=[nb, nb, nb,
                  pl.BlockSpec((NC, BR, H), lambda i: (0, i, 0)),
                  nb, n1, n1,
                  full3(Wxf), full3(Whf), full3(bgf),
                  full3(Wxb), full3(Whb), full3(bgb),
                  full3(fcf), full3(fcb), full3(W4),
                  pl.BlockSpec((1, O_PAD), lambda i: (0, 0))],
        out_specs=[pl.BlockSpec((BR, O_PAD), lambda i: (i, 0))] * 2,
        out_shape=[jax.ShapeDtypeStruct((NP, O_PAD), _F32)] * 2,
    )(p1, p2, p3, agg4p, hl4, dinv, sn,
      Wxf, Whf, bgf, Wxb, Whb, bgb, fcf, fcb, W4, b4.reshape(1, O_PAD))


def _final_call(agg5p, h5, dinv, sn):
    BR = 1024

    def body(p_ref, h_ref, dinv_ref, sn_ref, o_ref):
        o_ref[...] = (dinv_ref[...] * (p_ref[0] + p_ref[1])
                      + h_ref[...] * sn_ref[...])

    return pl.pallas_call(
        body,
        grid=(NP // BR,),
        in_specs=[pl.BlockSpec((NC, BR, O_PAD), lambda i: (0, i, 0)),
                  pl.BlockSpec((BR, O_PAD), lambda i: (i, 0)),
                  pl.BlockSpec((BR, 1), lambda i: (i, 0)),
                  pl.BlockSpec((BR, 1), lambda i: (i, 0))],
        out_specs=pl.BlockSpec((BR, O_PAD), lambda i: (i, 0)),
        out_shape=jax.ShapeDtypeStruct((NP, O_PAD), _F32),
    )(agg5p, h5, dinv, sn)


def _pad_gates(Wx, Wh, b):
    Wxg = jnp.stack([jnp.pad(Wx[:, g * U:(g + 1) * U], ((0, 0), (0, UP - U)))
                     for g in range(4)])
    Whg = jnp.stack([jnp.pad(Wh[:, g * U:(g + 1) * U],
                             ((0, UP - U), (0, UP - U)))
                     for g in range(4)])
    bg = jnp.stack([jnp.pad(b[g * U:(g + 1) * U], (0, UP - U))
                    for g in range(4)])
    return Wxg, Whg, bg


def kernel(x, edge_index, W0, b0, W1, b1, W2, b2, W3, b3,
           Wx_fw, Wh_fw, bl_fw, Wx_bw, Wh_bw, bl_bw, fc_fw, fc_bw, W4, b4):
    src, dst = edge_index[0], edge_index[1]
    src_p = jnp.concatenate(
        [src, jnp.zeros((E_PAD - E_EDGES,), jnp.int32)]).reshape(
            TOT_CH, CHUNK)
    dst_p = jnp.concatenate(
        [dst, jnp.full((E_PAD - E_EDGES,), N_NODES, jnp.int32)]).reshape(
            TOT_CH, CHUNK)
    xp = jnp.concatenate([x, jnp.zeros((NP - N_NODES, D_IN), x.dtype)])

    degp = _deg_call(dst_p)
    dinv, sn = _prep_call(degp)

    agg_h = _make_agg_call(H)
    hl = _l0_matmul(xp, W0, b0)
    hs = _scale_call(hl, dinv)
    posts = []
    for Wn, bn in ((W1, b1), (W2, b2), (W3, b3)):
        aggp = agg_h(hs, src_p, dst_p)
        post, hl, hs = _layer_call(aggp, hl, dinv, sn, Wn, bn)
        posts.append(post)
    agg4p = agg_h(hs, src_p, dst_p)

    Wxf, Whf, bgf = _pad_gates(Wx_fw, Wh_fw, bl_fw)
    Wxb, Whb, bgb = _pad_gates(Wx_bw, Wh_bw, bl_bw)
    fcf = jnp.pad(fc_fw[:, :, 0], ((0, 0), (0, UP - U)))
    fcb = jnp.pad(fc_bw[:, :, 0], ((0, 0), (0, UP - U)))

    W4p = jnp.pad(W4, ((0, 0), (0, O_PAD - O_DIM)))
    b4p = jnp.pad(b4, (0, O_PAD - O_DIM))
    h5, hs5 = _lstm_call(posts[0], posts[1], posts[2], agg4p, hl, dinv, sn,
                         Wxf, Whf, bgf, Wxb, Whb, bgb, fcf, fcb, W4p, b4p)

    agg5p = _make_agg_call(O_PAD)(hs5, src_p, dst_p)
    out = _final_call(agg5p, h5, dinv, sn)
    return out[:N_NODES, :O_DIM]
